# baseline (speedup 1.0000x reference)
import functools

import jax
import jax.numpy as jnp
import numpy as np
from jax import lax
from jax.experimental import pallas as pl
from jax.experimental.pallas import tpu as pltpu
from jax.experimental.pallas import tpu_sc as plsc

_NUM_TIME_MASKS = 10
_NUM_FREQ_MASKS = 2
_TIME_MASK_RATIO = 0.05
_MAX_FREQ_MASK_SIZE = 27

_NC, _NS, _L = 2, 16, 16


def _mask_constants(frame: int, n_mels: int):
    rng = np.random.default_rng(0)
    f = int(rng.integers(0, _MAX_FREQ_MASK_SIZE + 1))
    f0 = rng.integers(0, n_mels - f, size=(_NUM_FREQ_MASKS,))
    fcols = np.zeros((n_mels,), bool)
    if f > 0:
        for s in f0:
            fcols[s : s + f] = True
    max_t = int(np.floor(_TIME_MASK_RATIO * frame))
    t = int(rng.integers(0, max_t + 1))
    t0 = rng.integers(0, frame - t, size=(_NUM_TIME_MASKS,))
    segs = []
    if t > 0:
        for s in sorted(int(v) for v in t0):
            segs.append((s, s + t))
    runs, prev = [], 0
    for s, e in segs:
        if s > prev:
            runs.append((prev, s))
        prev = max(prev, e)
    if prev < frame:
        runs.append((prev, frame))
    return runs, segs, fcols


def kernel(x):
    b, frame, n_mels = x.shape
    _, segs, fcols = _mask_constants(frame, n_mels)
    nw = _NC * _NS
    bpw = b // nw

    zsegs = []
    for s, e in segs:
        a, z = -(-s // 8) * 8, (e // 8) * 8
        if a < z:
            zsegs.append((a, z))
    runs, prev = [], 0
    for a, z in zsegs:
        if a > prev:
            runs.append((prev, a))
        prev = z
    if prev < frame:
        runs.append((prev, frame))
    run_msubs = []
    for r0, r1 in runs:
        msubs = []
        for s, e in segs:
            a, bb_ = max(s, r0), min(e, r1)
            if a < bb_:
                msubs.append((a - r0, bb_ - r0))
        run_msubs.append(msubs)
    maxrun = max(r1 - r0 for r0, r1 in runs)
    zrows = max(z - a for a, z in zsegs) if zsegs else 8

    groups = []
    for g in range(n_mels // _L):
        lanes = fcols[g * _L : (g + 1) * _L]
        if lanes.any():
            idx = np.nonzero(lanes)[0]
            lo, hi = int(idx[0]), int(idx[-1]) + 1
            assert lanes[lo:hi].all(), "masked freq lanes must be contiguous"
            groups.append((g, lanes.all(), lo, hi))

    mesh = plsc.VectorSubcoreMesh(
        core_axis_name="c", subcore_axis_name="s",
        num_cores=_NC, num_subcores=_NS,
    )

    @functools.partial(
        pl.kernel,
        mesh=mesh,
        out_type=jax.ShapeDtypeStruct((b, frame, n_mels), jnp.float32),
        scratch_types=[
            pltpu.VMEM((2, maxrun, n_mels), jnp.float32),
            pltpu.VMEM((zrows, n_mels), jnp.float32),
            pltpu.SemaphoreType.DMA((2,)),
            pltpu.SemaphoreType.DMA((2,)),
            pltpu.SemaphoreType.DMA,
        ],
    )
    def sc_kernel(x_hbm, out_hbm, buf, zbuf, insems, outsems, zsem):
        wid = lax.axis_index("s") * _NC + lax.axis_index("c")
        base = wid * bpw
        z16 = jnp.zeros((_L,), jnp.float32)
        lane = lax.iota(jnp.int32, _L)

        def zrow(r, _):
            for g in range(n_mels // _L):
                zbuf[r, pl.ds(g * _L, _L)] = z16
            return _
        lax.fori_loop(0, zrows, zrow, None, unroll=False)

        for k in range(bpw):
            for s, e in zsegs:
                pltpu.make_async_copy(
                    zbuf.at[pl.ds(0, e - s), :],
                    out_hbm.at[base + k, pl.ds(s, e - s), :],
                    zsem,
                ).start()

        tasks = [
            (k, r0, r1, ri)
            for k in range(bpw)
            for ri, (r0, r1) in enumerate(runs)
        ]
        nt = len(tasks)

        def in_copy(t, slot):
            k, r0, r1, _ri = tasks[t]
            return pltpu.make_async_copy(
                x_hbm.at[base + k, pl.ds(r0, r1 - r0), :],
                buf.at[slot, pl.ds(0, r1 - r0), :],
                insems.at[slot],
            )

        def out_copy(t, slot):
            k, r0, r1, _ri = tasks[t]
            return pltpu.make_async_copy(
                buf.at[slot, pl.ds(0, r1 - r0), :],
                out_hbm.at[base + k, pl.ds(r0, r1 - r0), :],
                outsems.at[slot],
            )

        in_copy(0, 0).start()
        for t in range(nt):
            slot = t % 2
            if t + 1 < nt:
                nslot = (t + 1) % 2
                if t >= 1:
                    out_copy(t - 1, nslot).wait()
                in_copy(t + 1, nslot).start()
            in_copy(t, slot).wait()

            _, r0, r1, _ri2 = tasks[t]

            def frow(r, _):
                for g, full, lo, hi in groups:
                    if full:
                        buf[slot, r, pl.ds(g * _L, _L)] = z16
                    else:
                        v = buf[slot, r, pl.ds(g * _L, _L)]
                        lmask = (lane >= lo) & (lane < hi)
                        buf[slot, r, pl.ds(g * _L, _L)] = jnp.where(
                            lmask, 0.0, v
                        )
                return _
            lax.fori_loop(0, r1 - r0, frow, None, unroll=False)

            def zfull(r, _):
                for g in range(n_mels // _L):
                    buf[slot, r, pl.ds(g * _L, _L)] = z16
                return _
            ridx = tasks[t][3]
            for a, bb_ in run_msubs[ridx]:
                lax.fori_loop(a, bb_, zfull, None, unroll=False)

            out_copy(t, slot).start()

        out_copy(nt - 1, (nt - 1) % 2).wait()
        if nt >= 2:
            out_copy(nt - 2, (nt - 2) % 2).wait()
        for k in range(bpw):
            for s, e in zsegs:
                pltpu.make_async_copy(
                    zbuf.at[pl.ds(0, e - s), :],
                    out_hbm.at[base + k, pl.ds(s, e - s), :],
                    zsem,
                ).wait()

    return sc_kernel(x)


# speedup vs baseline: 1.6384x; 1.6384x over previous
import jax
import jax.numpy as jnp
import numpy as np
from jax.experimental import pallas as pl
from jax.experimental.pallas import tpu as pltpu

_NUM_TIME_MASKS = 10
_NUM_FREQ_MASKS = 2
_TIME_MASK_RATIO = 0.05
_MAX_FREQ_MASK_SIZE = 27

_BB = 8


def _mask_constants(frame: int, n_mels: int):
    rng = np.random.default_rng(0)
    f = int(rng.integers(0, _MAX_FREQ_MASK_SIZE + 1))
    f0 = rng.integers(0, n_mels - f, size=(_NUM_FREQ_MASKS,))
    fmask = np.ones((n_mels,), np.float32)
    if f > 0:
        for s in f0:
            fmask[s : s + f] = 0.0
    max_t = int(np.floor(_TIME_MASK_RATIO * frame))
    t = int(rng.integers(0, max_t + 1))
    t0 = rng.integers(0, frame - t, size=(_NUM_TIME_MASKS,))
    tmask = np.ones((frame,), np.float32)
    segs = []
    if t > 0:
        for s in sorted(int(v) for v in t0):
            tmask[s : s + t] = 0.0
            segs.append((s, s + t))
    runs, prev = [], 0
    for s, e in segs:
        if s > prev:
            runs.append((prev, s))
        prev = max(prev, e)
    if prev < frame:
        runs.append((prev, frame))
    plane = tmask[:, None] * fmask[None, :]
    return runs, segs, plane


def kernel(x):
    b, frame, n_mels = x.shape
    runs, segs, plane = _mask_constants(frame, n_mels)
    mask = jnp.asarray(plane)[None, :, :]
    nsteps = b // _BB
    nruns = len(runs)
    zmax = max(e - s for s, e in segs) if segs else 1

    def body(x_hbm, m_ref, o_hbm, buf, zbuf, isems, osems, zsem):
        i = pl.program_id(0)
        slot = jax.lax.rem(i, 3)

        def in_copy(step, slot, ridx):
            r0, r1 = runs[ridx]
            return pltpu.make_async_copy(
                x_hbm.at[pl.ds(step * _BB, _BB), pl.ds(r0, r1 - r0), :],
                buf.at[slot, :, pl.ds(r0, r1 - r0), :],
                isems.at[slot, ridx],
            )

        def out_copy(step, slot, ridx):
            r0, r1 = runs[ridx]
            return pltpu.make_async_copy(
                buf.at[slot, :, pl.ds(r0, r1 - r0), :],
                o_hbm.at[pl.ds(step * _BB, _BB), pl.ds(r0, r1 - r0), :],
                osems.at[slot, ridx],
            )

        def z_copy(step, sidx):
            s, e = segs[sidx]
            return pltpu.make_async_copy(
                zbuf.at[:, pl.ds(0, e - s), :],
                o_hbm.at[pl.ds(step * _BB, _BB), pl.ds(s, e - s), :],
                zsem,
            )

        @pl.when(i == 0)
        def _():
            zbuf[...] = jnp.zeros((_BB, zmax, n_mels), jnp.float32)
            for r in range(nruns):
                in_copy(0, 0, r).start()
            for r in range(nruns):
                in_copy(1, 1, r).start()
            for r in range(nruns):
                in_copy(2, 2, r).start()

        for sidx in range(len(segs)):
            z_copy(i, sidx).start()

        @pl.when((i >= 1) & (i + 2 < nsteps))
        def _():
            nslot = jax.lax.rem(i + 2, 3)
            for r in range(nruns):
                out_copy(i - 1, nslot, r).wait()
            for r in range(nruns):
                in_copy(i + 2, nslot, r).start()

        for ridx, (r0, r1) in enumerate(runs):
            in_copy(i, slot, ridx).wait()
            buf[slot, :, pl.ds(r0, r1 - r0), :] = jnp.where(
                m_ref[:, pl.ds(r0, r1 - r0), :] != 0.0,
                buf[slot, :, pl.ds(r0, r1 - r0), :],
                0.0,
            )
            out_copy(i, slot, ridx).start()

        for sidx in range(len(segs)):
            z_copy(i, sidx).wait()

        @pl.when(i == nsteps - 1)
        def _():
            for step in (nsteps - 3, nsteps - 2, nsteps - 1):
                for r in range(nruns):
                    out_copy(step, step % 3, r).wait()

    return pl.pallas_call(
        body,
        grid=(nsteps,),
        in_specs=[
            pl.BlockSpec(memory_space=pl.ANY),
            pl.BlockSpec((1, frame, n_mels), lambda i: (0, 0, 0)),
        ],
        out_specs=pl.BlockSpec(memory_space=pl.ANY),
        out_shape=jax.ShapeDtypeStruct(x.shape, x.dtype),
        scratch_shapes=[
            pltpu.VMEM((3, _BB, frame, n_mels), jnp.float32),
            pltpu.VMEM((_BB, zmax, n_mels), jnp.float32),
            pltpu.SemaphoreType.DMA((3, nruns)),
            pltpu.SemaphoreType.DMA((3, nruns)),
            pltpu.SemaphoreType.DMA,
        ],
    )(x, mask)


# speedup vs baseline: 1.6547x; 1.0099x over previous
import jax
import jax.numpy as jnp
import numpy as np
from jax.experimental import pallas as pl
from jax.experimental.pallas import tpu as pltpu

_NUM_TIME_MASKS = 10
_NUM_FREQ_MASKS = 2
_TIME_MASK_RATIO = 0.05
_MAX_FREQ_MASK_SIZE = 27

_BB = 16


def _mask_constants(frame: int, n_mels: int):
    rng = np.random.default_rng(0)
    f = int(rng.integers(0, _MAX_FREQ_MASK_SIZE + 1))
    f0 = rng.integers(0, n_mels - f, size=(_NUM_FREQ_MASKS,))
    fmask = np.ones((n_mels,), np.float32)
    if f > 0:
        for s in f0:
            fmask[s : s + f] = 0.0
    max_t = int(np.floor(_TIME_MASK_RATIO * frame))
    t = int(rng.integers(0, max_t + 1))
    t0 = rng.integers(0, frame - t, size=(_NUM_TIME_MASKS,))
    tmask = np.ones((frame,), np.float32)
    segs = []
    if t > 0:
        for s in sorted(int(v) for v in t0):
            tmask[s : s + t] = 0.0
            segs.append((s, s + t))
    runs, prev = [], 0
    for s, e in segs:
        if s > prev:
            runs.append((prev, s))
        prev = max(prev, e)
    if prev < frame:
        runs.append((prev, frame))
    plane = tmask[:, None] * fmask[None, :]
    return runs, segs, plane


def kernel(x):
    b, frame, n_mels = x.shape
    runs, segs, plane = _mask_constants(frame, n_mels)
    mask = jnp.asarray(plane)[None, :, :]
    nsteps = b // _BB
    nruns = len(runs)
    zmax = max(e - s for s, e in segs) if segs else 1

    def body(x_hbm, m_ref, o_hbm, buf, zbuf, isems, osems, zsem):
        i = pl.program_id(0)
        slot = jax.lax.rem(i, 3)

        def in_copy(step, slot, ridx):
            r0, r1 = runs[ridx]
            return pltpu.make_async_copy(
                x_hbm.at[pl.ds(step * _BB, _BB), pl.ds(r0, r1 - r0), :],
                buf.at[slot, :, pl.ds(r0, r1 - r0), :],
                isems.at[slot, ridx],
            )

        def out_copy(step, slot, ridx):
            r0, r1 = runs[ridx]
            return pltpu.make_async_copy(
                buf.at[slot, :, pl.ds(r0, r1 - r0), :],
                o_hbm.at[pl.ds(step * _BB, _BB), pl.ds(r0, r1 - r0), :],
                osems.at[slot, ridx],
            )

        def z_copy(step, sidx):
            s, e = segs[sidx]
            return pltpu.make_async_copy(
                zbuf.at[:, pl.ds(0, e - s), :],
                o_hbm.at[pl.ds(step * _BB, _BB), pl.ds(s, e - s), :],
                zsem,
            )

        @pl.when(i == 0)
        def _():
            zbuf[...] = jnp.zeros((_BB, zmax, n_mels), jnp.float32)
            for r in range(nruns):
                in_copy(0, 0, r).start()
            for r in range(nruns):
                in_copy(1, 1, r).start()
            for r in range(nruns):
                in_copy(2, 2, r).start()

        for sidx in range(len(segs)):
            z_copy(i, sidx).start()

        @pl.when((i >= 1) & (i + 2 < nsteps))
        def _():
            nslot = jax.lax.rem(i + 2, 3)
            for r in range(nruns):
                out_copy(i - 1, nslot, r).wait()
            for r in range(nruns):
                in_copy(i + 2, nslot, r).start()

        for ridx, (r0, r1) in enumerate(runs):
            in_copy(i, slot, ridx).wait()
            buf[slot, :, pl.ds(r0, r1 - r0), :] = jnp.where(
                m_ref[:, pl.ds(r0, r1 - r0), :] != 0.0,
                buf[slot, :, pl.ds(r0, r1 - r0), :],
                0.0,
            )
            out_copy(i, slot, ridx).start()

        for sidx in range(len(segs)):
            z_copy(i, sidx).wait()

        @pl.when(i == nsteps - 1)
        def _():
            for step in (nsteps - 3, nsteps - 2, nsteps - 1):
                for r in range(nruns):
                    out_copy(step, step % 3, r).wait()

    return pl.pallas_call(
        body,
        grid=(nsteps,),
        in_specs=[
            pl.BlockSpec(memory_space=pl.ANY),
            pl.BlockSpec((1, frame, n_mels), lambda i: (0, 0, 0)),
        ],
        out_specs=pl.BlockSpec(memory_space=pl.ANY),
        out_shape=jax.ShapeDtypeStruct(x.shape, x.dtype),
        scratch_shapes=[
            pltpu.VMEM((3, _BB, frame, n_mels), jnp.float32),
            pltpu.VMEM((_BB, zmax, n_mels), jnp.float32),
            pltpu.SemaphoreType.DMA((3, nruns)),
            pltpu.SemaphoreType.DMA((3, nruns)),
            pltpu.SemaphoreType.DMA,
        ],
    )(x, mask)


# speedup vs baseline: 1.6697x; 1.0091x over previous
import jax
import jax.numpy as jnp
import numpy as np
from jax.experimental import pallas as pl
from jax.experimental.pallas import tpu as pltpu

_NUM_TIME_MASKS = 10
_NUM_FREQ_MASKS = 2
_TIME_MASK_RATIO = 0.05
_MAX_FREQ_MASK_SIZE = 27

_BB = 8


def _mask_constants(frame: int, n_mels: int):
    rng = np.random.default_rng(0)
    f = int(rng.integers(0, _MAX_FREQ_MASK_SIZE + 1))
    f0 = rng.integers(0, n_mels - f, size=(_NUM_FREQ_MASKS,))
    fmask = np.ones((n_mels,), np.float32)
    if f > 0:
        for s in f0:
            fmask[s : s + f] = 0.0
    max_t = int(np.floor(_TIME_MASK_RATIO * frame))
    t = int(rng.integers(0, max_t + 1))
    t0 = rng.integers(0, frame - t, size=(_NUM_TIME_MASKS,))
    tmask = np.ones((frame,), np.float32)
    segs = []
    if t > 0:
        for s in sorted(int(v) for v in t0):
            tmask[s : s + t] = 0.0
            segs.append((s, s + t))
    runs, prev = [], 0
    for s, e in segs:
        if s > prev:
            runs.append((prev, s))
        prev = max(prev, e)
    if prev < frame:
        runs.append((prev, frame))
    plane = tmask[:, None] * fmask[None, :]
    return runs, segs, plane


def kernel(x):
    b, frame, n_mels = x.shape
    runs, segs, plane = _mask_constants(frame, n_mels)
    mask = jnp.asarray(plane)[None, :, :]
    nsteps = b // _BB
    nruns = len(runs)

    def body(x_hbm, m_ref, o_ref, buf, sems):
        i = pl.program_id(0)
        slot = jax.lax.rem(i, 4)

        def copy(step, slot, ridx):
            r0, r1 = runs[ridx]
            return pltpu.make_async_copy(
                x_hbm.at[pl.ds(step * _BB, _BB), pl.ds(r0, r1 - r0), :],
                buf.at[slot, :, pl.ds(r0, r1 - r0), :],
                sems.at[slot, ridx],
            )

        @pl.when(i == 0)
        def _():
            for s_ in range(3):
                for r in range(nruns):
                    copy(s_, s_, r).start()

        @pl.when(i + 3 < nsteps)
        def _():
            for r in range(nruns):
                copy(i + 3, jax.lax.rem(i + 3, 4), r).start()

        for m0, m1 in segs:
            o_ref[:, pl.ds(m0, m1 - m0), :] = jnp.zeros(
                (_BB, m1 - m0, n_mels), jnp.float32
            )
        for ridx, (r0, r1) in enumerate(runs):
            copy(i, slot, ridx).wait()
            o_ref[:, pl.ds(r0, r1 - r0), :] = jnp.where(
                m_ref[:, pl.ds(r0, r1 - r0), :] != 0.0,
                buf[slot, :, pl.ds(r0, r1 - r0), :],
                0.0,
            )

    return pl.pallas_call(
        body,
        grid=(nsteps,),
        in_specs=[
            pl.BlockSpec(memory_space=pl.ANY),
            pl.BlockSpec((1, frame, n_mels), lambda i: (0, 0, 0)),
        ],
        out_specs=pl.BlockSpec((_BB, frame, n_mels), lambda i: (i, 0, 0)),
        out_shape=jax.ShapeDtypeStruct(x.shape, x.dtype),
        scratch_shapes=[
            pltpu.VMEM((4, _BB, frame, n_mels), jnp.float32),
            pltpu.SemaphoreType.DMA((4, nruns)),
        ],
    )(x, mask)
